# Initial kernel scaffold; baseline (speedup 1.0000x reference)
import jax
import jax.numpy as jnp
from jax.experimental import pallas as pl


def kernel(x):
    raise NotImplementedError("write your pallas kernel here")


# speedup vs baseline: 4.4082x; 4.4082x over previous
import jax
import jax.numpy as jnp
from jax.experimental import pallas as pl
from jax.experimental.pallas import tpu as pltpu

_WIDTH = 512
_NUM_COLS = 2
_SCALE = 0.001
_BLOCK_ROWS = 4096


def _scores() -> jnp.ndarray:
    xs = jnp.linspace(-15.0, 15.0, _WIDTH)
    a = 0.0014888176096
    b = 0.0
    c = 0.0152831145355
    parabola = a * (xs - b) ** 2 + c
    parabola = parabola / parabola.sum()
    gkey = jax.random.key(42)
    u = jax.random.uniform(gkey, (_WIDTH,), minval=1e-10, maxval=1.0)
    gumbel = -jnp.log(-jnp.log(u))
    return jnp.log(parabola) + gumbel


def _body(scores_ref, x_ref, o_ref):
    s = scores_ref[0, :]
    lane = jax.lax.broadcasted_iota(jnp.int32, (1, _WIDTH), 1)[0]
    m1 = jnp.max(s)
    i1 = jnp.min(jnp.where(s == m1, lane, _WIDTH))
    s2 = jnp.where(lane == i1, -jnp.inf, s)
    m2 = jnp.max(s2)
    i2 = jnp.min(jnp.where(s2 == m2, lane, _WIDTH))
    erased = (lane == i1) | (lane == i2)
    mask = jnp.where(erased, jnp.float32(_SCALE), jnp.float32(1.0))
    o_ref[...] = x_ref[...] * mask[None, :]


def kernel(x):
    n, c, h, w = x.shape
    rows = n * c * h
    x2 = x.reshape(rows, w)
    grid = rows // _BLOCK_ROWS
    out = pl.pallas_call(
        _body,
        grid=(grid,),
        in_specs=[
            pl.BlockSpec((1, _WIDTH), lambda i: (0, 0)),
            pl.BlockSpec((_BLOCK_ROWS, _WIDTH), lambda i: (i, 0)),
        ],
        out_specs=pl.BlockSpec((_BLOCK_ROWS, _WIDTH), lambda i: (i, 0)),
        out_shape=jax.ShapeDtypeStruct((rows, w), x.dtype),
    )(_scores().reshape(1, _WIDTH), x2)
    return out.reshape(n, c, h, w)
